# baseline (speedup 1.0000x reference)
import functools

import jax
import jax.numpy as jnp
from jax import lax
from jax.experimental import pallas as pl
from jax.experimental.pallas import tpu as pltpu
from jax.experimental.pallas import tpu_sc as plsc

NUM_EMB = 49
EMBED = 64
NC = 2
NS = 16
NW = NC * NS
L = 16

CHUNK = 512
GROUPS = CHUNK // L


@functools.partial(jax.jit, static_argnames=("b_tot",))
def _sc_embedding_lookup(idx_flat, table_flat, *, b_tot):
    per_w = b_tot // NW
    n_chunks = per_w // CHUNK
    n_outer = n_chunks // 2

    mesh = plsc.VectorSubcoreMesh(core_axis_name="c", subcore_axis_name="s")

    @functools.partial(
        pl.kernel,
        mesh=mesh,
        compiler_params=pltpu.CompilerParams(use_tc_tiling_on_sc=False, needs_layout_passes=False),
        out_type=jax.ShapeDtypeStruct((b_tot * EMBED,), jnp.float32),
        scratch_types=dict(
            idx_v=pltpu.VMEM((2, CHUNK), jnp.int32),
            rows_v=pltpu.VMEM((2, CHUNK * EMBED), jnp.float32),
            table_v=pltpu.VMEM((NUM_EMB * EMBED,), jnp.float32),
            sem_i0=pltpu.SemaphoreType.DMA,
            sem_i1=pltpu.SemaphoreType.DMA,
            sem_w0=pltpu.SemaphoreType.DMA,
            sem_w1=pltpu.SemaphoreType.DMA,
        ),
    )
    def k(idx_hbm, table_hbm, out_hbm, idx_v, rows_v, table_v,
          sem_i0, sem_i1, sem_w0, sem_w1):
        wid = lax.axis_index("s") * NC + lax.axis_index("c")
        base = wid * per_w
        sem_i = (sem_i0, sem_i1)
        sem_w = (sem_w0, sem_w1)
        pltpu.sync_copy(table_hbm, table_v)
        lane64 = lax.iota(jnp.int32, L) * 64

        def load_idx(ch, slot):
            pltpu.async_copy(
                idx_hbm.at[pl.ds(base + ch * CHUNK, CHUNK)],
                idx_v.at[slot],
                sem_i[slot],
            )

        def drain_idx(slot):
            pltpu.make_async_copy(
                idx_hbm.at[pl.ds(base, CHUNK)], idx_v.at[slot], sem_i[slot]
            ).wait()

        def compute_chunk(slot):
            def group(g, _):
                scaled = idx_v[slot, pl.ds(g * L, L)] * 64
                obase = lane64 + g * (L * EMBED)

                def col(c, _):
                    vals = plsc.load_gather(table_v, [scaled + c])
                    plsc.store_scatter(rows_v.at[slot], [obase + c], vals)
                    return 0

                lax.fori_loop(0, EMBED, col, 0, unroll=8)
                return 0

            lax.fori_loop(0, GROUPS, group, 0, unroll=False)

        def fire_write(ch, slot):
            pltpu.async_copy(
                rows_v.at[slot],
                out_hbm.at[pl.ds((base + ch * CHUNK) * EMBED, CHUNK * EMBED)],
                sem_w[slot],
            )

        def drain_write(slot):
            pltpu.make_async_copy(
                rows_v.at[slot],
                out_hbm.at[pl.ds(base * EMBED, CHUNK * EMBED)],
                sem_w[slot],
            ).wait()

        def step(ch, slot, prefetch, drain_w):
            if drain_w:
                drain_write(slot)
            compute_chunk(slot)
            fire_write(ch, slot)
            if prefetch:
                load_idx(ch + 2, slot)

        load_idx(0, 0)
        load_idx(1, 1)
        drain_idx(0)
        step(0, 0, True, False)
        drain_idx(1)
        step(1, 1, True, False)
        drain_idx(0)

        def outer(i, carry):
            ch = i * 2
            step(ch, 0, True, True)
            drain_idx(1)
            step(ch + 1, 1, True, True)
            drain_idx(0)
            return carry

        lax.fori_loop(1, n_outer - 1, outer, 0, unroll=False)

        ch = (n_outer - 1) * 2
        step(ch, 0, False, True)
        drain_idx(1)
        step(ch + 1, 1, False, True)
        drain_write(0)
        drain_write(1)

    return k(idx_flat, table_flat)


def kernel(time, table):
    b, h = time.shape
    idx_flat = time.reshape(b * h).astype(jnp.int32)
    out = _sc_embedding_lookup(
        idx_flat, table.reshape(NUM_EMB * EMBED), b_tot=b * h
    )
    return out.reshape(b, h, EMBED)


# speedup vs baseline: 4.0771x; 4.0771x over previous
import functools

import jax
import jax.numpy as jnp
from jax import lax
from jax.experimental import pallas as pl
from jax.experimental.pallas import tpu as pltpu
from jax.experimental.pallas import tpu_sc as plsc

NUM_EMB = 49
EMBED = 64
NC = 2
NS = 16
NW = NC * NS

CHUNK = 128
BLOCK = 5


@functools.partial(jax.jit, static_argnames=("b_tot",))
def _sc_embedding_lookup(idx2d, table, *, b_tot):
    rows_tot = b_tot // CHUNK
    rows_per_w = rows_tot // NW
    n_blocks = rows_per_w // BLOCK
    n_outer = n_blocks // 2

    mesh = plsc.VectorSubcoreMesh(core_axis_name="c", subcore_axis_name="s")

    @functools.partial(
        pl.kernel,
        mesh=mesh,
        compiler_params=pltpu.CompilerParams(use_tc_tiling_on_sc=False),
        out_type=jax.ShapeDtypeStruct((b_tot, EMBED), jnp.float32),
        scratch_types=dict(
            idx_v=pltpu.VMEM((2, BLOCK, CHUNK), jnp.int32),
            rows_v=pltpu.VMEM((2, BLOCK, CHUNK, EMBED), jnp.float32),
            table_v=pltpu.VMEM_SHARED((NS * NUM_EMB, EMBED), jnp.float32),
            sem_i=pltpu.SemaphoreType.DMA,
            sem_g=pltpu.SemaphoreType.DMA,
            sem_w=pltpu.SemaphoreType.DMA,
        ),
    )
    def k(idx_hbm, table_hbm, out_hbm, idx_v, rows_v, table_v,
          sem_i, sem_g, sem_w):
        wid = lax.axis_index("s") * NC + lax.axis_index("c")
        base_row = wid * rows_per_w
        @pl.when(lax.axis_index("s") == 0)
        def _():
            pltpu.sync_copy(table_hbm, table_v)
        plsc.subcore_barrier()

        def load_idx(blk, slot):
            row0 = base_row + blk * BLOCK
            pltpu.async_copy(
                idx_hbm.at[pl.ds(row0, BLOCK), :], idx_v.at[slot], sem_i
            )

        def drain_idx(slot):
            pltpu.make_async_copy(
                idx_hbm.at[pl.ds(base_row, BLOCK), :], idx_v.at[slot], sem_i
            ).wait()

        def fire_gathers(slot):
            for j in range(BLOCK):
                pltpu.async_copy(
                    table_v.at[idx_v.at[slot, j]], rows_v.at[slot, j], sem_g
                )

        def fire_writes(blk, slot):
            row0 = base_row + blk * BLOCK
            for j in range(BLOCK):
                pltpu.make_async_copy(
                    table_v.at[idx_v.at[slot, j]], rows_v.at[slot, j], sem_g
                ).wait()
                pltpu.async_copy(
                    rows_v.at[slot, j],
                    out_hbm.at[pl.ds((row0 + j) * CHUNK, CHUNK)],
                    sem_w,
                )

        def drain_writes(blk, slot):
            row0 = base_row + blk * BLOCK
            for j in range(BLOCK):
                pltpu.make_async_copy(
                    rows_v.at[slot, j],
                    out_hbm.at[pl.ds((row0 + j) * CHUNK, CHUNK)],
                    sem_w,
                ).wait()

        def step(blk, slot, prefetch):
            other = 1 - slot
            fire_writes(blk, slot)
            drain_idx(other)
            if prefetch:
                load_idx(blk + 2, slot)
            fire_gathers(other)
            drain_writes(blk, slot)

        load_idx(0, 0)
        drain_idx(0)
        load_idx(1, 1)
        fire_gathers(0)

        def outer(i, carry):
            blk = i * 2
            step(blk, 0, True)
            step(blk + 1, 1, True)
            return carry

        lax.fori_loop(0, n_outer - 1, outer, 0, unroll=False)

        blk = (n_outer - 1) * 2
        step(blk, 0, False)
        fire_writes(blk + 1, 1)
        drain_writes(blk + 1, 1)

    return k(idx2d, table)


def kernel(time, table):
    b, h = time.shape
    b_tot = b * h
    idx = time.reshape(b_tot).astype(jnp.int32)
    per_w = b_tot // NW
    sub = (jnp.arange(b_tot, dtype=jnp.int32) // per_w) // NC
    idx2d = (idx + NUM_EMB * sub).reshape(b_tot // CHUNK, CHUNK)
    table_rep = jnp.tile(table, (NS, 1))
    out = _sc_embedding_lookup(idx2d, table_rep, b_tot=b_tot)
    return out.reshape(b, h, EMBED)
